# baseline (speedup 1.0000x reference)
import functools

import jax
import jax.numpy as jnp
from jax.experimental import pallas as pl
from jax.experimental.pallas import tpu as pltpu

_VOCAB = 100000
_TOPK = 100
_VBLK = 2048


def _scores_body(pooled_ref, emb_ref, out_ref):
    j = pl.program_id(0)
    scores = jax.lax.dot_general(
        pooled_ref[...], emb_ref[...],
        dimension_numbers=(((1,), (1,)), ((), ())),
        preferred_element_type=jnp.float32,
    )
    col = j * _VBLK + jax.lax.broadcasted_iota(jnp.int32, scores.shape, 1)
    out_ref[...] = jnp.where(col < _VOCAB, scores, -jnp.inf)


def _scores(pooled, emb_padded, n_blocks):
    b, d = pooled.shape
    return pl.pallas_call(
        _scores_body,
        grid=(n_blocks,),
        in_specs=[
            pl.BlockSpec((b, d), lambda j: (0, 0)),
            pl.BlockSpec((_VBLK, d), lambda j: (j, 0)),
        ],
        out_specs=pl.BlockSpec((b, _VBLK), lambda j: (0, j)),
        out_shape=jax.ShapeDtypeStruct((b, n_blocks * _VBLK), jnp.float32),
    )(pooled, emb_padded)


@jax.jit
def kernel(item_seq, item_seq_len, embedding):
    b, hist = item_seq.shape
    v, d = embedding.shape
    seq_emb = jnp.take(embedding, item_seq, axis=0)
    lens = jnp.maximum(item_seq_len, 1)
    pos = jnp.arange(hist)[None, :]
    mask = (pos < lens[:, None]).astype(seq_emb.dtype)
    pooled = jnp.sum(seq_emb * mask[:, :, None], axis=1) / lens[:, None].astype(
        seq_emb.dtype)

    n_blocks = (v + _VBLK - 1) // _VBLK
    pad = n_blocks * _VBLK - v
    emb_padded = jnp.pad(embedding, ((0, pad), (0, 0)))
    scores = _scores(pooled, emb_padded, n_blocks)
    values, indices = jax.lax.top_k(scores, _TOPK)
    return values, indices


# speedup vs baseline: 10.7045x; 10.7045x over previous
import functools

import jax
import jax.numpy as jnp
from jax import lax
from jax.experimental import pallas as pl
from jax.experimental.pallas import tpu as pltpu
from jax.experimental.pallas import tpu_sc as plsc

_VOCAB = 100000
_TOPK = 100
_VBLK = 2048
_NBLK = 49
_VPAD = _NBLK * _VBLK
_BATCH = 1024

_W = 12544
_NWIN = _VPAD // _W
_CAP = 512
_TRIG = 384
_NCAND = 128


def _scores_body(pooled_ref, emb_ref, out_ref):
    j = pl.program_id(0)
    scores = lax.dot_general(
        pooled_ref[...], emb_ref[...],
        dimension_numbers=(((1,), (1,)), ((), ())),
        preferred_element_type=jnp.float32,
    )
    col = j * _VBLK + lax.broadcasted_iota(jnp.int32, scores.shape, 1)
    out_ref[...] = jnp.where(col < _VOCAB, scores, -jnp.inf)


def _scores(pooled, emb_padded):
    b, d = pooled.shape
    return pl.pallas_call(
        _scores_body,
        grid=(_NBLK,),
        in_specs=[
            pl.BlockSpec((b, d), lambda j: (0, 0)),
            pl.BlockSpec((_VBLK, d), lambda j: (j, 0)),
        ],
        out_specs=pl.BlockSpec((b, _VBLK), lambda j: (0, j)),
        out_shape=jax.ShapeDtypeStruct((b, _VPAD), jnp.float32),
    )(pooled, emb_padded)


def _f2key(x):
    ub = plsc.bitcast(x, jnp.uint32)
    return jnp.where(ub >> 31 == 1, ~ub, ub | jnp.uint32(0x80000000))


def _key2f(tau, t_old):
    bits = jnp.where(tau >= jnp.uint32(0x80000000),
                     tau & jnp.uint32(0x7FFFFFFF), ~tau)
    t_f = lax.bitcast_convert_type(bits, jnp.float32)
    ok = (t_f == t_f) & (t_f > t_old)
    return jnp.where(ok, t_f, t_old)


def _sc_topk_body(scores_hbm, outv_hbm, outi_hbm, wbuf, bufs, bufi, kbuf):
    info = plsc.get_sparse_core_info()
    nc, ns = info.num_cores, info.num_subcores
    wid = lax.axis_index("s") * nc + lax.axis_index("c")
    rows_per_w = _BATCH // (nc * ns)
    iota16 = lax.iota(jnp.int32, 16)
    nv = _CAP // 16

    def count_ge(tau):
        cnt = jnp.int32(0)
        for i in range(nv):
            kv = kbuf[pl.ds(16 * i, 16)]
            cnt = cnt + jnp.sum((kv >= tau).astype(jnp.int32))
        return cnt

    def count_gt(tau):
        cnt = jnp.int32(0)
        for i in range(nv):
            kv = kbuf[pl.ds(16 * i, 16)]
            cnt = cnt + jnp.sum((kv > tau).astype(jnp.int32))
        return cnt

    def build_kbuf(n):
        for i in range(nv):
            s = bufs[pl.ds(16 * i, 16)]
            valid = (16 * i + iota16) < n
            kbuf[pl.ds(16 * i, 16)] = jnp.where(valid, _f2key(s), jnp.uint32(0))

    def value_search_exact(accept):
        def body(it, tau):
            b = (31 - it).astype(jnp.uint32)
            cand = tau | (jnp.uint32(1) << b)
            return jnp.where(count_ge(cand) >= accept, cand, tau)
        return lax.fori_loop(0, 32, body, jnp.uint32(0))

    def cut_exact_128(tau):
        k2 = jnp.int32(128) - count_gt(tau)

        def ibit(it, ti):
            b = 16 - it
            upper = ti | ((jnp.int32(1) << b) - 1)
            cnt = jnp.int32(0)
            for i in range(nv):
                kv = kbuf[pl.ds(16 * i, 16)]
                ix = bufi[pl.ds(16 * i, 16)]
                m = (kv == tau) & (ix <= upper)
                cnt = cnt + jnp.sum(m.astype(jnp.int32))
            return jnp.where(cnt >= k2, ti, ti | (jnp.int32(1) << b))

        ti = lax.fori_loop(0, 17, ibit, jnp.int32(0))
        n2 = jnp.int32(0)
        for i in range(nv):
            kv = kbuf[pl.ds(16 * i, 16)]
            ix = bufi[pl.ds(16 * i, 16)]
            keep = (kv > tau) | ((kv == tau) & (ix <= ti))
            s = bufs[pl.ds(16 * i, 16)]
            c = plsc.cumsum(jnp.where(keep, jnp.int32(1), jnp.int32(0)))
            pos = (n2 - 1) + c
            plsc.store_scatter(bufs, [pos], s, mask=keep)
            plsc.store_scatter(bufi, [pos], ix, mask=keep)
            n2 = n2 + jnp.sum(keep.astype(jnp.int32))
        return n2

    def prune_inloop(nt):
        n, t_old = nt
        build_kbuf(n)
        tau = value_search_exact(jnp.int32(128))
        n2 = cut_exact_128(tau)
        return n2, _key2f(tau, t_old)

    def row_body(r, _):
        row = wid * rows_per_w + r

        def win_body(w, nt):
            pltpu.sync_copy(scores_hbm.at[row, pl.ds(w * _W, _W)], wbuf)
            col0 = w * _W

            def group_body(g, nt):
                n, t = nt
                base = g * 128
                xs = [wbuf[pl.ds(base + 16 * v, 16)] for v in range(8)]
                anym = xs[0] > t
                for v in range(1, 8):
                    anym = anym | (xs[v] > t)

                def do_insert(nt):
                    n, t = lax.cond(nt[0] > _TRIG, prune_inloop,
                                    lambda c: c, nt)
                    for v in range(8):
                        x = xs[v]
                        m = x > t
                        c = plsc.cumsum(jnp.where(m, jnp.int32(1),
                                                  jnp.int32(0)))
                        pos = (n - 1) + c
                        gi = col0 + base + 16 * v + iota16
                        plsc.store_scatter(bufs, [pos], x, mask=m)
                        plsc.store_scatter(bufi, [pos], gi, mask=m)
                        n = n + jnp.sum(m.astype(jnp.int32))
                    return n, t

                return lax.cond(jnp.any(anym), do_insert, lambda c: c, (n, t))

            return lax.fori_loop(0, _W // 128, group_body, nt)

        n, t = lax.fori_loop(0, _NWIN, win_body,
                             (jnp.int32(0), jnp.float32(-jnp.inf)))
        build_kbuf(n)
        tau = value_search_exact(jnp.int32(128))
        cut_exact_128(tau)
        pltpu.sync_copy(bufs.at[pl.ds(0, _NCAND)], outv_hbm.at[row])
        pltpu.sync_copy(bufi.at[pl.ds(0, _NCAND)], outi_hbm.at[row])
        return 0

    lax.fori_loop(0, rows_per_w, row_body, 0)


def _sc_topk(scores):
    mesh = plsc.VectorSubcoreMesh(core_axis_name="c", subcore_axis_name="s")
    f = pl.kernel(
        _sc_topk_body,
        out_type=[
            jax.ShapeDtypeStruct((_BATCH, _NCAND), jnp.float32),
            jax.ShapeDtypeStruct((_BATCH, _NCAND), jnp.int32),
        ],
        mesh=mesh,
        scratch_types=[
            pltpu.VMEM((_W,), jnp.float32),
            pltpu.VMEM((_CAP,), jnp.float32),
            pltpu.VMEM((_CAP,), jnp.int32),
            pltpu.VMEM((_CAP,), jnp.uint32),
        ],
        compiler_params=pltpu.CompilerParams(needs_layout_passes=False),
    )
    return f(scores)


def _sort_body(v_ref, i_ref, outv_ref, outi_ref):
    v = v_ref[...]
    ix = i_ref[...]
    b, n = v.shape
    lane = lax.broadcasted_iota(jnp.int32, (b, n), 1)
    k = 2
    while k <= n:
        j = k // 2
        while j >= 1:
            vl = pltpu.roll(v, n - j, 1)
            vr = pltpu.roll(v, j, 1)
            il = pltpu.roll(ix, n - j, 1)
            ir = pltpu.roll(ix, j, 1)
            low = (lane & j) == 0
            pv = jnp.where(low, vl, vr)
            pi = jnp.where(low, il, ir)
            dsc = (lane & k) == 0
            self_first = (v > pv) | ((v == pv) & (ix < pi))
            take_self = self_first == (low == dsc)
            v = jnp.where(take_self, v, pv)
            ix = jnp.where(take_self, ix, pi)
            j //= 2
        k *= 2
    outv_ref[...] = v[:, :_TOPK]
    outi_ref[...] = ix[:, :_TOPK]


def _final_sort(vals, idxs):
    b = vals.shape[0]
    return pl.pallas_call(
        _sort_body,
        out_shape=[
            jax.ShapeDtypeStruct((b, _TOPK), jnp.float32),
            jax.ShapeDtypeStruct((b, _TOPK), jnp.int32),
        ],
    )(vals, idxs)


@jax.jit
def kernel(item_seq, item_seq_len, embedding):
    b, hist = item_seq.shape
    v, d = embedding.shape
    seq_emb = jnp.take(embedding, item_seq, axis=0)
    lens = jnp.maximum(item_seq_len, 1)
    pos = jnp.arange(hist)[None, :]
    mask = (pos < lens[:, None]).astype(seq_emb.dtype)
    pooled = jnp.sum(seq_emb * mask[:, :, None], axis=1) / lens[:, None].astype(
        seq_emb.dtype)

    emb_padded = jnp.pad(embedding, ((0, _VPAD - v), (0, 0)))
    scores = _scores(pooled, emb_padded)
    cand_v, cand_i = _sc_topk(scores)
    values, indices = _final_sort(cand_v, cand_i)
    return values, indices
